# Initial kernel scaffold; baseline (speedup 1.0000x reference)
import jax
import jax.numpy as jnp
from jax.experimental import pallas as pl


def kernel(x, mask, freqs_cis, w_router, wq, wk, wv, wo, w1, w2, w3, g1, g2):
    raise NotImplementedError("write your pallas kernel here")


# speedup vs baseline: 1.7972x; 1.7972x over previous
import functools

import jax
import jax.numpy as jnp
import numpy as np
from jax.experimental import pallas as pl
from jax.experimental.pallas import tpu as pltpu


def _router_body(x_ref, w_ref, o_ref):
    o_ref[...] = jnp.dot(x_ref[...], w_ref[...],
                         preferred_element_type=jnp.float32)


def _router_logits(x2d, w_pad):
    n, d = x2d.shape
    bm = min(512, n)
    return pl.pallas_call(
        _router_body,
        grid=(n // bm,),
        in_specs=[
            pl.BlockSpec((bm, d), lambda i: (i, 0)),
            pl.BlockSpec((d, 128), lambda i: (0, 0)),
        ],
        out_specs=pl.BlockSpec((bm, 128), lambda i: (i, 0)),
        out_shape=jax.ShapeDtypeStruct((n, 128), jnp.float32),
    )(x2d, w_pad)


def _qkv_body(x_ref, g_ref, w_ref, o_ref):
    xb = x_ref[0]
    ms = jnp.mean(xb * xb, axis=1, keepdims=True)
    h = xb * jax.lax.rsqrt(ms + 1e-5) * g_ref[...]
    o_ref[0] = jnp.dot(h.astype(jnp.bfloat16), w_ref[...],
                       preferred_element_type=jnp.float32).astype(jnp.bfloat16)


def _qkv(filt, g1, wqkv, bm):
    b, k, d = filt.shape
    return pl.pallas_call(
        _qkv_body,
        grid=(b, k // bm, 3),
        in_specs=[
            pl.BlockSpec((1, bm, d), lambda i, j, n: (i, j, 0)),
            pl.BlockSpec((1, d), lambda i, j, n: (0, 0)),
            pl.BlockSpec((d, d), lambda i, j, n: (0, n)),
        ],
        out_specs=pl.BlockSpec((1, bm, d), lambda i, j, n: (i, j, n)),
        out_shape=jax.ShapeDtypeStruct((b, k, 3 * d), jnp.bfloat16),
    )(filt, g1, wqkv)


def _attn_body(q_ref, k_ref, v_ref, cs_ref, o_ref, *, hd):
    half = hd // 2
    c = cs_ref[:, :half]
    s = cs_ref[:, half:]
    q = q_ref[0].astype(jnp.float32)
    k = k_ref[0].astype(jnp.float32)
    q1, q2 = q[:, :half], q[:, half:]
    k1, k2 = k[:, :half], k[:, half:]
    qr = jnp.concatenate([q1 * c - q2 * s, q1 * s + q2 * c], axis=1)
    kr = jnp.concatenate([k1 * c - k2 * s, k1 * s + k2 * c], axis=1)
    scores = jax.lax.dot_general(
        qr.astype(jnp.bfloat16), kr.astype(jnp.bfloat16),
        (((1,), (1,)), ((), ())),
        preferred_element_type=jnp.float32) * (1.0 / np.sqrt(hd))
    m = jnp.max(scores, axis=1, keepdims=True)
    e = jnp.exp(scores - m)
    a = e / jnp.sum(e, axis=1, keepdims=True)
    o_ref[0] = jnp.dot(a.astype(jnp.bfloat16), v_ref[0],
                       preferred_element_type=jnp.float32).astype(jnp.bfloat16)


def _attention(qkv, cs, nh, hd):
    b, k, _ = qkv.shape
    d = nh * hd
    return pl.pallas_call(
        functools.partial(_attn_body, hd=hd),
        grid=(b, nh),
        in_specs=[
            pl.BlockSpec((1, k, hd), lambda i, h: (i, 0, h)),
            pl.BlockSpec((1, k, hd), lambda i, h: (i, 0, nh + h)),
            pl.BlockSpec((1, k, hd), lambda i, h: (i, 0, 2 * nh + h)),
            pl.BlockSpec((k, hd), lambda i, h: (0, 0)),
        ],
        out_specs=pl.BlockSpec((1, k, hd), lambda i, h: (i, 0, h)),
        out_shape=jax.ShapeDtypeStruct((b, k, d), jnp.bfloat16),
    )(qkv, qkv, qkv, cs)


def _wo_body(a_ref, f_ref, w_ref, o_ref):
    o_ref[0] = f_ref[0] + jnp.dot(a_ref[0], w_ref[...],
                                  preferred_element_type=jnp.float32)


def _wo_residual(attn, filt, wo_b, bm, bn):
    b, k, d = filt.shape
    return pl.pallas_call(
        _wo_body,
        grid=(b, k // bm, d // bn),
        in_specs=[
            pl.BlockSpec((1, bm, d), lambda i, j, n: (i, j, 0)),
            pl.BlockSpec((1, bm, bn), lambda i, j, n: (i, j, n)),
            pl.BlockSpec((d, bn), lambda i, j, n: (0, n)),
        ],
        out_specs=pl.BlockSpec((1, bm, bn), lambda i, j, n: (i, j, n)),
        out_shape=jax.ShapeDtypeStruct((b, k, d), jnp.float32),
    )(attn, filt, wo_b)


def _mlp_body(x1_ref, ft_ref, rw_ref, g_ref, w1_ref, w3_ref, w2_ref, o_ref,
              *, nf):
    f = pl.program_id(2)
    x1 = x1_ref[0]
    ms = jnp.mean(x1 * x1, axis=1, keepdims=True)
    h2 = (x1 * jax.lax.rsqrt(ms + 1e-5) * g_ref[...]).astype(jnp.bfloat16)
    a1 = jnp.dot(h2, w1_ref[...], preferred_element_type=jnp.float32)
    a3 = jnp.dot(h2, w3_ref[...], preferred_element_type=jnp.float32)
    gate = (a1 * jax.nn.sigmoid(a1) * a3).astype(jnp.bfloat16)
    delta = jnp.dot(gate, w2_ref[...], preferred_element_type=jnp.float32)

    @pl.when(f == 0)
    def _():
        o_ref[0] = delta

    @pl.when(f > 0)
    def _():
        o_ref[0] = o_ref[0] + delta

    @pl.when(f == nf - 1)
    def _():
        o_ref[0] = ft_ref[0] + rw_ref[0][:, :1] * (x1 + o_ref[0])


def _mlp_combine(x1, filt, rwb, g2, w1_b, w3_b, w2_b, bm, bf):
    b, k, d = x1.shape
    dff = w1_b.shape[1]
    nf = dff // bf
    return pl.pallas_call(
        functools.partial(_mlp_body, nf=nf),
        grid=(b, k // bm, nf),
        in_specs=[
            pl.BlockSpec((1, bm, d), lambda i, j, n: (i, j, 0)),
            pl.BlockSpec((1, bm, d), lambda i, j, n: (i, j, 0)),
            pl.BlockSpec((1, bm, 128), lambda i, j, n: (i, j, 0)),
            pl.BlockSpec((1, d), lambda i, j, n: (0, 0)),
            pl.BlockSpec((d, bf), lambda i, j, n: (0, n)),
            pl.BlockSpec((d, bf), lambda i, j, n: (0, n)),
            pl.BlockSpec((bf, d), lambda i, j, n: (n, 0)),
        ],
        out_specs=pl.BlockSpec((1, bm, d), lambda i, j, n: (i, j, 0)),
        out_shape=jax.ShapeDtypeStruct((b, k, d), jnp.float32),
        compiler_params=pltpu.CompilerParams(
            dimension_semantics=("parallel", "parallel", "arbitrary")),
    )(x1, filt, rwb, g2, w1_b, w3_b, w2_b)


def kernel(x, mask, freqs_cis, w_router, wq, wk, wv, wo, w1, w2, w3, g1, g2):
    b, seq, d = x.shape
    k, half = freqs_cis.shape
    hd = 2 * half
    nh = d // hd
    dff = w1.shape[1]
    bm = min(256, k)

    w_pad = jnp.pad(w_router, ((0, 0), (0, 127)))
    logits = _router_logits(x.reshape(b * seq, d), w_pad)[:, 0].reshape(b, seq)

    _, token_index = jax.lax.top_k(logits, k)
    selected = jnp.sort(token_index, axis=1)
    sel_logits = jnp.take_along_axis(logits, selected, axis=1)
    r_weights = jax.nn.softmax(sel_logits, axis=1)

    filt = jnp.take_along_axis(x, selected[:, :, None], axis=1)

    def rope_perm(w):
        return (w.reshape(d, nh, half, 2).transpose(0, 1, 3, 2)
                .reshape(d, d))

    wqkv = jnp.concatenate(
        [rope_perm(wq), rope_perm(wk), wv], axis=1).astype(jnp.bfloat16)
    cs = jnp.concatenate([jnp.cos(freqs_cis), jnp.sin(freqs_cis)], axis=1)

    qkv = _qkv(filt, g1.reshape(1, d), wqkv, bm)
    attn = _attention(qkv, cs, nh, hd)
    x1 = _wo_residual(attn, filt, wo.astype(jnp.bfloat16), bm, min(1024, d))
    rwb = jnp.broadcast_to(r_weights[:, :, None], (b, k, 128))
    frows = _mlp_combine(x1, filt, rwb, g2.reshape(1, d),
                         w1.astype(jnp.bfloat16), w3.astype(jnp.bfloat16),
                         w2.astype(jnp.bfloat16), bm, min(1024, dff))

    out = x.at[jnp.arange(b)[:, None], selected].set(
        frows, unique_indices=True, indices_are_sorted=True)
    return out
